# Initial kernel scaffold; baseline (speedup 1.0000x reference)
import jax
import jax.numpy as jnp
from jax.experimental import pallas as pl


def kernel(x, weight, log_sigma, gamma, beta, noise):
    raise NotImplementedError("write your pallas kernel here")


# speedup vs baseline: 3.2405x; 3.2405x over previous
import functools

import jax
import jax.numpy as jnp
from jax.experimental import pallas as pl
from jax.experimental.pallas import tpu as pltpu

_VAR_EPS = 1e-8
_BN_EPS = 1e-5
_HALO = 128


def _conv_kernel(xa_ref, xb_ref, xc_ref, wm_ref, wv_ref, noise_ref,
                 y_ref, sum_ref, sq_ref, p_ref, p2_ref,
                 *, tm, kh, kw, c_in, ho_dim, wo_dim):
    i = pl.program_id(0)

    xwin = jnp.concatenate(
        [xa_ref[:, tm - _HALO:], xb_ref[...], xc_ref[:, :_HALO]], axis=1)
    p = jnp.maximum(xwin, 0)

    pos = jax.lax.broadcasted_iota(jnp.int32, (1, tm), 1) + i * tm
    wo = pos % wo_dim
    hos = (pos // wo_dim) % ho_dim
    one = jnp.ones((1, tm), jnp.float32)
    zero = jnp.zeros((1, tm), jnp.float32)

    def _mask(cond):
        return jnp.where(cond, one, zero).astype(jnp.bfloat16)

    col_m = {0: _mask(wo >= 1), 1: None, 2: _mask(wo <= wo_dim - 2)}
    row_m = {0: _mask(hos >= 1), 1: None, 2: _mask(hos <= ho_dim - 2)}

    for di in range(kh):
        for dj in range(kw):
            t = di * kw + dj
            off = _HALO + (di - 1) * wo_dim + (dj - 1)
            tap = p[:, off:off + tm]
            if row_m[di] is not None and col_m[dj] is not None:
                tap = tap * (row_m[di] * col_m[dj])
            elif row_m[di] is not None:
                tap = tap * row_m[di]
            elif col_m[dj] is not None:
                tap = tap * col_m[dj]
            p_ref[t * c_in:(t + 1) * c_in, :] = tap

    pv = p_ref[...]
    p2_ref[...] = pv * pv
    mu = jnp.dot(wm_ref[...], p_ref[...], preferred_element_type=jnp.float32)
    var = jnp.dot(wv_ref[...], p2_ref[...], preferred_element_type=jnp.float32)

    y = mu + jnp.sqrt(_VAR_EPS + var) * noise_ref[...]
    y_ref[...] = y
    sum_ref[0] = jnp.sum(y, axis=1, keepdims=True)
    sq_ref[0] = jnp.sum(y * y, axis=1, keepdims=True)


def _bn_kernel(y_ref, scale_ref, shift_ref, o_ref):
    o_ref[...] = y_ref[...] * scale_ref[...] + shift_ref[...]


def _pick_tile(m, cap):
    for cand in (cap, cap // 2, cap // 4, cap // 8, 128):
        if cand >= 128 and m % cand == 0:
            return cand
    return m


@functools.partial(jax.jit, static_argnames=())
def kernel(x, weight, log_sigma, gamma, beta, noise):
    n, c_in, h, w = x.shape
    c_out, _, kh, kw = weight.shape
    ho, wo = h, w
    m = n * ho * wo
    k = c_in * kh * kw

    xt = x.transpose(1, 0, 2, 3).reshape(c_in, m).astype(jnp.bfloat16)
    noise_t = noise.transpose(1, 0, 2, 3).reshape(c_out, m).astype(jnp.float32)
    wm = weight.transpose(0, 2, 3, 1).reshape(c_out, k).astype(jnp.bfloat16)
    wv = (0.01 + jnp.exp(2.0 * log_sigma.astype(jnp.float32)))
    wv = wv.transpose(0, 2, 3, 1).reshape(c_out, k).astype(jnp.bfloat16)

    tm = _pick_tile(m, 2048)
    n_tiles = m // tm
    xp = jnp.pad(xt, ((0, 0), (tm, tm)))

    grid = (n_tiles,)
    y_t, psum, psq = pl.pallas_call(
        functools.partial(_conv_kernel, tm=tm, kh=kh, kw=kw, c_in=c_in,
                          ho_dim=ho, wo_dim=wo),
        grid=grid,
        in_specs=[
            pl.BlockSpec((c_in, tm), lambda i: (0, i)),
            pl.BlockSpec((c_in, tm), lambda i: (0, i + 1)),
            pl.BlockSpec((c_in, tm), lambda i: (0, i + 2)),
            pl.BlockSpec((c_out, k), lambda i: (0, 0)),
            pl.BlockSpec((c_out, k), lambda i: (0, 0)),
            pl.BlockSpec((c_out, tm), lambda i: (0, i)),
        ],
        out_specs=[
            pl.BlockSpec((c_out, tm), lambda i: (0, i)),
            pl.BlockSpec((1, c_out, 1), lambda i: (i, 0, 0)),
            pl.BlockSpec((1, c_out, 1), lambda i: (i, 0, 0)),
        ],
        out_shape=(
            jax.ShapeDtypeStruct((c_out, m), jnp.float32),
            jax.ShapeDtypeStruct((n_tiles, c_out, 1), jnp.float32),
            jax.ShapeDtypeStruct((n_tiles, c_out, 1), jnp.float32),
        ),
        scratch_shapes=[
            pltpu.VMEM((k, tm), jnp.bfloat16),
            pltpu.VMEM((k, tm), jnp.bfloat16),
        ],
        compiler_params=pltpu.CompilerParams(
            dimension_semantics=("parallel",),
            vmem_limit_bytes=100 * 1024 * 1024,
        ),
    )(xp, xp, xp, wm, wv, noise_t)

    total = jnp.sum(psum, axis=0)[:, 0]
    total_sq = jnp.sum(psq, axis=0)[:, 0]
    mean = total / m
    var = total_sq / m - mean * mean
    inv = gamma / jnp.sqrt(var + _BN_EPS)
    scale = inv.reshape(c_out, 1).astype(jnp.float32)
    shift = (beta - mean * inv).reshape(c_out, 1).astype(jnp.float32)

    tb = _pick_tile(m, 4096)
    out_t = pl.pallas_call(
        _bn_kernel,
        grid=(m // tb,),
        in_specs=[
            pl.BlockSpec((c_out, tb), lambda i: (0, i)),
            pl.BlockSpec((c_out, 1), lambda i: (0, 0)),
            pl.BlockSpec((c_out, 1), lambda i: (0, 0)),
        ],
        out_specs=pl.BlockSpec((c_out, tb), lambda i: (0, i)),
        out_shape=jax.ShapeDtypeStruct((c_out, m), jnp.float32),
        compiler_params=pltpu.CompilerParams(
            dimension_semantics=("parallel",),
        ),
    )(y_t, scale, shift)

    return out_t.reshape(c_out, n, ho, wo).transpose(1, 0, 2, 3)


# speedup vs baseline: 3.7083x; 1.1443x over previous
import functools

import jax
import jax.numpy as jnp
from jax.experimental import pallas as pl
from jax.experimental.pallas import tpu as pltpu

_VAR_EPS = 1e-8
_BN_EPS = 1e-5
_HALO = 128


def _conv_kernel(xa_ref, xb_ref, xc_ref, wm_ref, wv_ref, noise_ref,
                 y_ref, sum_ref, sq_ref, p_ref, p2_ref,
                 *, tm, kh, kw, c_in, ho_dim, wo_dim):
    j = pl.program_id(1)

    xwin = jnp.concatenate(
        [xa_ref[0, :, tm - _HALO:], xb_ref[0], xc_ref[0, :, :_HALO]], axis=1)
    p = jnp.maximum(xwin, 0).astype(jnp.bfloat16)

    pos = jax.lax.broadcasted_iota(jnp.int32, (1, tm), 1) + j * tm
    wo = pos % wo_dim
    hos = pos // wo_dim
    one = jnp.ones((1, tm), jnp.float32)
    zero = jnp.zeros((1, tm), jnp.float32)

    def _mask(cond):
        return jnp.where(cond, one, zero).astype(jnp.bfloat16)

    col_m = {0: _mask(wo >= 1), 1: None, 2: _mask(wo <= wo_dim - 2)}
    row_m = {0: _mask(hos >= 1), 1: None, 2: _mask(hos <= ho_dim - 2)}

    for di in range(kh):
        for dj in range(kw):
            t = di * kw + dj
            off = _HALO + (di - 1) * wo_dim + (dj - 1)
            tap = p[:, off:off + tm]
            if row_m[di] is not None and col_m[dj] is not None:
                tap = tap * (row_m[di] * col_m[dj])
            elif row_m[di] is not None:
                tap = tap * row_m[di]
            elif col_m[dj] is not None:
                tap = tap * col_m[dj]
            p_ref[t * c_in:(t + 1) * c_in, :] = tap

    pv = p_ref[...]
    p2_ref[...] = pv * pv
    mu = jnp.dot(wm_ref[...], p_ref[...], preferred_element_type=jnp.float32)
    var = jnp.dot(wv_ref[...], p2_ref[...], preferred_element_type=jnp.float32)

    y = mu + jnp.sqrt(_VAR_EPS + var) * noise_ref[0]
    y_ref[0] = y
    sum_ref[0] = jnp.sum(y, axis=1, keepdims=True)
    sq_ref[0] = jnp.sum(y * y, axis=1, keepdims=True)


def _bn_kernel(y_ref, scale_ref, shift_ref, o_ref):
    o_ref[0] = y_ref[0] * scale_ref[...] + shift_ref[...]


def _pick_tile(m, cap):
    for cand in (cap, cap // 2, cap // 4, cap // 8, 128):
        if cand >= 128 and m % cand == 0:
            return cand
    return m


def kernel(x, weight, log_sigma, gamma, beta, noise):
    n, c_in, h, w = x.shape
    c_out, _, kh, kw = weight.shape
    ho, wo = h, w
    hw = ho * wo
    m = n * hw
    k = c_in * kh * kw

    x3 = x.reshape(n, c_in, hw)
    noise3 = noise.reshape(n, c_out, hw)
    wm = weight.transpose(0, 2, 3, 1).reshape(c_out, k).astype(jnp.bfloat16)
    wv = (0.01 + jnp.exp(2.0 * log_sigma.astype(jnp.float32)))
    wv = wv.transpose(0, 2, 3, 1).reshape(c_out, k).astype(jnp.bfloat16)

    tm = _pick_tile(hw, 2048)
    tpi = hw // tm
    n_tiles = n * tpi

    y3, psum, psq = pl.pallas_call(
        functools.partial(_conv_kernel, tm=tm, kh=kh, kw=kw, c_in=c_in,
                          ho_dim=ho, wo_dim=wo),
        grid=(n, tpi),
        in_specs=[
            pl.BlockSpec((1, c_in, tm),
                         lambda i, j: (i, 0, jnp.maximum(j - 1, 0))),
            pl.BlockSpec((1, c_in, tm), lambda i, j: (i, 0, j)),
            pl.BlockSpec((1, c_in, tm),
                         lambda i, j: (i, 0, jnp.minimum(j + 1, tpi - 1))),
            pl.BlockSpec((c_out, k), lambda i, j: (0, 0)),
            pl.BlockSpec((c_out, k), lambda i, j: (0, 0)),
            pl.BlockSpec((1, c_out, tm), lambda i, j: (i, 0, j)),
        ],
        out_specs=[
            pl.BlockSpec((1, c_out, tm), lambda i, j: (i, 0, j)),
            pl.BlockSpec((1, c_out, 1), lambda i, j: (i * tpi + j, 0, 0)),
            pl.BlockSpec((1, c_out, 1), lambda i, j: (i * tpi + j, 0, 0)),
        ],
        out_shape=(
            jax.ShapeDtypeStruct((n, c_out, hw), jnp.float32),
            jax.ShapeDtypeStruct((n_tiles, c_out, 1), jnp.float32),
            jax.ShapeDtypeStruct((n_tiles, c_out, 1), jnp.float32),
        ),
        scratch_shapes=[
            pltpu.VMEM((k, tm), jnp.bfloat16),
            pltpu.VMEM((k, tm), jnp.bfloat16),
        ],
        compiler_params=pltpu.CompilerParams(
            dimension_semantics=("parallel", "parallel"),
            vmem_limit_bytes=100 * 1024 * 1024,
        ),
    )(x3, x3, x3, wm, wv, noise3)

    total = jnp.sum(psum, axis=0)[:, 0]
    total_sq = jnp.sum(psq, axis=0)[:, 0]
    mean = total / m
    var = total_sq / m - mean * mean
    inv = gamma / jnp.sqrt(var + _BN_EPS)
    scale = inv.reshape(c_out, 1).astype(jnp.float32)
    shift = (beta - mean * inv).reshape(c_out, 1).astype(jnp.float32)

    tb = _pick_tile(hw, 4096)
    out3 = pl.pallas_call(
        _bn_kernel,
        grid=(n, hw // tb),
        in_specs=[
            pl.BlockSpec((1, c_out, tb), lambda i, j: (i, 0, j)),
            pl.BlockSpec((c_out, 1), lambda i, j: (0, 0)),
            pl.BlockSpec((c_out, 1), lambda i, j: (0, 0)),
        ],
        out_specs=pl.BlockSpec((1, c_out, tb), lambda i, j: (i, 0, j)),
        out_shape=jax.ShapeDtypeStruct((n, c_out, hw), jnp.float32),
        compiler_params=pltpu.CompilerParams(
            dimension_semantics=("parallel", "parallel"),
        ),
    )(y3, scale, shift)

    return out3.reshape(n, c_out, ho, wo)


# speedup vs baseline: 3.9561x; 1.0668x over previous
import functools

import jax
import jax.numpy as jnp
from jax.experimental import pallas as pl
from jax.experimental.pallas import tpu as pltpu

_VAR_EPS = 1e-8
_BN_EPS = 1e-5
_HALO = 128


def _conv_kernel(*refs, tm, kh, kw, c_in, ho_dim, wo_dim, single_tile):
    if single_tile:
        (xb_ref, wm_ref, wv_ref, noise_ref,
         y_ref, sum_ref, sq_ref, p_ref, p2_ref) = refs
        xa_ref = xc_ref = xb_ref
    else:
        (xa_ref, xb_ref, xc_ref, wm_ref, wv_ref, noise_ref,
         y_ref, sum_ref, sq_ref, p_ref, p2_ref) = refs
    j = pl.program_id(1)

    xwin = jnp.concatenate(
        [xa_ref[0, :, tm - _HALO:], xb_ref[0], xc_ref[0, :, :_HALO]], axis=1)
    p = jnp.maximum(xwin, 0).astype(jnp.bfloat16)

    pos = jax.lax.broadcasted_iota(jnp.int32, (1, tm), 1) + j * tm
    wo = pos % wo_dim
    hos = pos // wo_dim
    one = jnp.ones((1, tm), jnp.float32)
    zero = jnp.zeros((1, tm), jnp.float32)

    def _mask(cond):
        return jnp.where(cond, one, zero).astype(jnp.bfloat16)

    col_m = {0: _mask(wo >= 1), 1: None, 2: _mask(wo <= wo_dim - 2)}
    row_m = {0: _mask(hos >= 1), 1: None, 2: _mask(hos <= ho_dim - 2)}

    for di in range(kh):
        for dj in range(kw):
            t = di * kw + dj
            off = _HALO + (di - 1) * wo_dim + (dj - 1)
            tap = p[:, off:off + tm]
            if row_m[di] is not None and col_m[dj] is not None:
                tap = tap * (row_m[di] * col_m[dj])
            elif row_m[di] is not None:
                tap = tap * row_m[di]
            elif col_m[dj] is not None:
                tap = tap * col_m[dj]
            p_ref[t * c_in:(t + 1) * c_in, :] = tap

    pv = p_ref[...]
    p2_ref[...] = pv * pv
    mu = jnp.dot(wm_ref[...], p_ref[...], preferred_element_type=jnp.float32)
    var = jnp.dot(wv_ref[...], p2_ref[...], preferred_element_type=jnp.float32)

    y = mu + jnp.sqrt(_VAR_EPS + var) * noise_ref[0]
    y_ref[0] = y
    sum_ref[0] = jnp.sum(y, axis=1, keepdims=True)
    sq_ref[0] = jnp.sum(y * y, axis=1, keepdims=True)


def _bn_kernel(y_ref, scale_ref, shift_ref, o_ref):
    o_ref[...] = y_ref[...] * scale_ref[...] + shift_ref[...]


def _pick_tile(m, cap):
    for cand in (cap, cap // 2, cap // 4, cap // 8, 128):
        if cand >= 128 and m % cand == 0:
            return cand
    return m


def kernel(x, weight, log_sigma, gamma, beta, noise):
    n, c_in, h, w = x.shape
    c_out, _, kh, kw = weight.shape
    ho, wo = h, w
    hw = ho * wo
    m = n * hw
    k = c_in * kh * kw

    x3 = x.reshape(n, c_in, hw)
    noise3 = noise.reshape(n, c_out, hw)
    wm = weight.transpose(0, 2, 3, 1).reshape(c_out, k).astype(jnp.bfloat16)
    wv = (0.01 + jnp.exp(2.0 * log_sigma.astype(jnp.float32)))
    wv = wv.transpose(0, 2, 3, 1).reshape(c_out, k).astype(jnp.bfloat16)

    tm = _pick_tile(hw, 4096)
    tpi = hw // tm
    n_tiles = n * tpi
    single = tpi == 1

    x_specs = [pl.BlockSpec((1, c_in, tm), lambda i, j: (i, 0, j))]
    x_args = [x3]
    if not single:
        x_specs = [
            pl.BlockSpec((1, c_in, tm),
                         lambda i, j: (i, 0, jnp.maximum(j - 1, 0))),
            pl.BlockSpec((1, c_in, tm), lambda i, j: (i, 0, j)),
            pl.BlockSpec((1, c_in, tm),
                         lambda i, j: (i, 0, jnp.minimum(j + 1, tpi - 1))),
        ]
        x_args = [x3, x3, x3]

    y3, psum, psq = pl.pallas_call(
        functools.partial(_conv_kernel, tm=tm, kh=kh, kw=kw, c_in=c_in,
                          ho_dim=ho, wo_dim=wo, single_tile=single),
        grid=(n, tpi),
        in_specs=x_specs + [
            pl.BlockSpec((c_out, k), lambda i, j: (0, 0)),
            pl.BlockSpec((c_out, k), lambda i, j: (0, 0)),
            pl.BlockSpec((1, c_out, tm), lambda i, j: (i, 0, j)),
        ],
        out_specs=[
            pl.BlockSpec((1, c_out, tm), lambda i, j: (i, 0, j)),
            pl.BlockSpec((1, c_out, 1), lambda i, j: (i * tpi + j, 0, 0)),
            pl.BlockSpec((1, c_out, 1), lambda i, j: (i * tpi + j, 0, 0)),
        ],
        out_shape=(
            jax.ShapeDtypeStruct((n, c_out, hw), jnp.float32),
            jax.ShapeDtypeStruct((n_tiles, c_out, 1), jnp.float32),
            jax.ShapeDtypeStruct((n_tiles, c_out, 1), jnp.float32),
        ),
        scratch_shapes=[
            pltpu.VMEM((k, tm), jnp.bfloat16),
            pltpu.VMEM((k, tm), jnp.bfloat16),
        ],
        compiler_params=pltpu.CompilerParams(
            dimension_semantics=("parallel", "parallel"),
            vmem_limit_bytes=100 * 1024 * 1024,
        ),
    )(*x_args, wm, wv, noise3)

    total = jnp.sum(psum, axis=0)[:, 0]
    total_sq = jnp.sum(psq, axis=0)[:, 0]
    mean = total / m
    var = total_sq / m - mean * mean
    inv = gamma / jnp.sqrt(var + _BN_EPS)
    scale = inv.reshape(c_out, 1).astype(jnp.float32)
    shift = (beta - mean * inv).reshape(c_out, 1).astype(jnp.float32)

    tb = _pick_tile(hw, 4096)
    bn = 2 if (n % 2 == 0 and tb == hw) else 1
    scale3 = scale.reshape(1, c_out, 1)
    shift3 = shift.reshape(1, c_out, 1)
    out3 = pl.pallas_call(
        _bn_kernel,
        grid=(n // bn, hw // tb),
        in_specs=[
            pl.BlockSpec((bn, c_out, tb), lambda i, j: (i, 0, j)),
            pl.BlockSpec((1, c_out, 1), lambda i, j: (0, 0, 0)),
            pl.BlockSpec((1, c_out, 1), lambda i, j: (0, 0, 0)),
        ],
        out_specs=pl.BlockSpec((bn, c_out, tb), lambda i, j: (i, 0, j)),
        out_shape=jax.ShapeDtypeStruct((n, c_out, hw), jnp.float32),
        compiler_params=pltpu.CompilerParams(
            dimension_semantics=("parallel", "parallel"),
        ),
    )(y3, scale3, shift3)

    return out3.reshape(n, c_out, ho, wo)


# speedup vs baseline: 4.0782x; 1.0309x over previous
import functools

import jax
import jax.numpy as jnp
from jax.experimental import pallas as pl
from jax.experimental.pallas import tpu as pltpu

_VAR_EPS = 1e-8
_BN_EPS = 1e-5
_HALO = 128


def _conv_kernel(*refs, tm, kh, kw, c_in, ho_dim, wo_dim, single_tile):
    if single_tile:
        (xb_ref, wm_ref, wv_ref, noise_ref,
         y_ref, sum_ref, sq_ref, p_ref, p2_ref) = refs
        xa_ref = xc_ref = xb_ref
    else:
        (xa_ref, xb_ref, xc_ref, wm_ref, wv_ref, noise_ref,
         y_ref, sum_ref, sq_ref, p_ref, p2_ref) = refs
    j = pl.program_id(1)

    xwin = jnp.concatenate(
        [xa_ref[0, :, tm - _HALO:], xb_ref[0], xc_ref[0, :, :_HALO]], axis=1)
    p = jnp.maximum(xwin, 0).astype(jnp.bfloat16)

    pos = jax.lax.broadcasted_iota(jnp.int32, (1, tm), 1) + j * tm
    wo = pos % wo_dim
    hos = pos // wo_dim
    one = jnp.ones((1, tm), jnp.float32)
    zero = jnp.zeros((1, tm), jnp.float32)

    def _mask(cond):
        return jnp.where(cond, one, zero).astype(jnp.bfloat16)

    col_m = {0: _mask(wo >= 1), 1: None, 2: _mask(wo <= wo_dim - 2)}
    row_m = {0: _mask(hos >= 1), 1: None, 2: _mask(hos <= ho_dim - 2)}

    for di in range(kh):
        for dj in range(kw):
            t = di * kw + dj
            off = _HALO + (di - 1) * wo_dim + (dj - 1)
            tap = p[:, off:off + tm]
            if row_m[di] is not None and col_m[dj] is not None:
                tap = tap * (row_m[di] * col_m[dj])
            elif row_m[di] is not None:
                tap = tap * row_m[di]
            elif col_m[dj] is not None:
                tap = tap * col_m[dj]
            p_ref[t * c_in:(t + 1) * c_in, :] = tap

    pv = p_ref[...]
    p2_ref[...] = pv * pv
    mu = jnp.dot(wm_ref[...], p_ref[...], preferred_element_type=jnp.float32)
    var = jnp.dot(wv_ref[...], p2_ref[...], preferred_element_type=jnp.float32)

    y = mu + jnp.sqrt(_VAR_EPS + var) * noise_ref[0]
    yb = y.astype(jnp.bfloat16)
    y_ref[0] = yb
    y32 = yb.astype(jnp.float32)
    sum_ref[0] = jnp.sum(y32, axis=1, keepdims=True)
    sq_ref[0] = jnp.sum(y32 * y32, axis=1, keepdims=True)


def _bn_kernel(y_ref, scale_ref, shift_ref, o_ref):
    o_ref[...] = (y_ref[...].astype(jnp.float32) * scale_ref[...]
                  + shift_ref[...])


def _pick_tile(m, cap):
    for cand in (cap, cap // 2, cap // 4, cap // 8, 128):
        if cand >= 128 and m % cand == 0:
            return cand
    return m


def kernel(x, weight, log_sigma, gamma, beta, noise):
    n, c_in, h, w = x.shape
    c_out, _, kh, kw = weight.shape
    ho, wo = h, w
    hw = ho * wo
    m = n * hw
    k = c_in * kh * kw

    x3 = x.reshape(n, c_in, hw)
    noise3 = noise.reshape(n, c_out, hw)
    wm = weight.transpose(0, 2, 3, 1).reshape(c_out, k).astype(jnp.bfloat16)
    wv = (0.01 + jnp.exp(2.0 * log_sigma.astype(jnp.float32)))
    wv = wv.transpose(0, 2, 3, 1).reshape(c_out, k).astype(jnp.bfloat16)

    tm = _pick_tile(hw, 4096)
    tpi = hw // tm
    n_tiles = n * tpi
    single = tpi == 1

    x_specs = [pl.BlockSpec((1, c_in, tm), lambda i, j: (i, 0, j))]
    x_args = [x3]
    if not single:
        x_specs = [
            pl.BlockSpec((1, c_in, tm),
                         lambda i, j: (i, 0, jnp.maximum(j - 1, 0))),
            pl.BlockSpec((1, c_in, tm), lambda i, j: (i, 0, j)),
            pl.BlockSpec((1, c_in, tm),
                         lambda i, j: (i, 0, jnp.minimum(j + 1, tpi - 1))),
        ]
        x_args = [x3, x3, x3]

    y3, psum, psq = pl.pallas_call(
        functools.partial(_conv_kernel, tm=tm, kh=kh, kw=kw, c_in=c_in,
                          ho_dim=ho, wo_dim=wo, single_tile=single),
        grid=(n, tpi),
        in_specs=x_specs + [
            pl.BlockSpec((c_out, k), lambda i, j: (0, 0)),
            pl.BlockSpec((c_out, k), lambda i, j: (0, 0)),
            pl.BlockSpec((1, c_out, tm), lambda i, j: (i, 0, j)),
        ],
        out_specs=[
            pl.BlockSpec((1, c_out, tm), lambda i, j: (i, 0, j)),
            pl.BlockSpec((1, c_out, 1), lambda i, j: (i * tpi + j, 0, 0)),
            pl.BlockSpec((1, c_out, 1), lambda i, j: (i * tpi + j, 0, 0)),
        ],
        out_shape=(
            jax.ShapeDtypeStruct((n, c_out, hw), jnp.bfloat16),
            jax.ShapeDtypeStruct((n_tiles, c_out, 1), jnp.float32),
            jax.ShapeDtypeStruct((n_tiles, c_out, 1), jnp.float32),
        ),
        scratch_shapes=[
            pltpu.VMEM((k, tm), jnp.bfloat16),
            pltpu.VMEM((k, tm), jnp.bfloat16),
        ],
        compiler_params=pltpu.CompilerParams(
            dimension_semantics=("parallel", "parallel"),
            vmem_limit_bytes=100 * 1024 * 1024,
        ),
    )(*x_args, wm, wv, noise3)

    total = jnp.sum(psum, axis=0)[:, 0]
    total_sq = jnp.sum(psq, axis=0)[:, 0]
    mean = total / m
    var = total_sq / m - mean * mean
    inv = gamma / jnp.sqrt(var + _BN_EPS)
    scale = inv.reshape(c_out, 1).astype(jnp.float32)
    shift = (beta - mean * inv).reshape(c_out, 1).astype(jnp.float32)

    tb = _pick_tile(hw, 4096)
    bn = 2 if (n % 2 == 0 and tb == hw) else 1
    scale3 = scale.reshape(1, c_out, 1)
    shift3 = shift.reshape(1, c_out, 1)
    out3 = pl.pallas_call(
        _bn_kernel,
        grid=(n // bn, hw // tb),
        in_specs=[
            pl.BlockSpec((bn, c_out, tb), lambda i, j: (i, 0, j)),
            pl.BlockSpec((1, c_out, 1), lambda i, j: (0, 0, 0)),
            pl.BlockSpec((1, c_out, 1), lambda i, j: (0, 0, 0)),
        ],
        out_specs=pl.BlockSpec((bn, c_out, tb), lambda i, j: (i, 0, j)),
        out_shape=jax.ShapeDtypeStruct((n, c_out, hw), jnp.float32),
        compiler_params=pltpu.CompilerParams(
            dimension_semantics=("parallel", "parallel"),
        ),
    )(y3, scale3, shift3)

    return out3.reshape(n, c_out, ho, wo)


# speedup vs baseline: 4.2001x; 1.0299x over previous
import functools

import jax
import jax.numpy as jnp
from jax.experimental import pallas as pl
from jax.experimental.pallas import tpu as pltpu

_VAR_EPS = 1e-8
_BN_EPS = 1e-5
_HALO = 128


def _fused_kernel(x_ref, wm_ref, wv_ref, noise_ref, g_ref, b_ref,
                  o_ref, y_scr, p_ref, p2_ref, sum_scr, sq_scr,
                  scale_scr, shift_scr,
                  *, n_img, kh, kw, c_in, ho_dim, wo_dim):
    hw = ho_dim * wo_dim
    i = pl.program_id(0)

    @pl.when(i < n_img)
    def _compute_phase():
        xb = x_ref[0]
        xwin = jnp.concatenate(
            [xb[:, hw - _HALO:], xb, xb[:, :_HALO]], axis=1)
        p = jnp.maximum(xwin, 0).astype(jnp.bfloat16)

        pos = jax.lax.broadcasted_iota(jnp.int32, (1, hw), 1)
        wo = pos % wo_dim
        hos = pos // wo_dim
        one = jnp.ones((1, hw), jnp.float32)
        zero = jnp.zeros((1, hw), jnp.float32)

        def _mask(cond):
            return jnp.where(cond, one, zero).astype(jnp.bfloat16)

        col_m = {0: _mask(wo >= 1), 1: None, 2: _mask(wo <= wo_dim - 2)}
        row_m = {0: _mask(hos >= 1), 1: None, 2: _mask(hos <= ho_dim - 2)}

        for di in range(kh):
            for dj in range(kw):
                t = di * kw + dj
                off = _HALO + (di - 1) * wo_dim + (dj - 1)
                tap = p[:, off:off + hw]
                if row_m[di] is not None and col_m[dj] is not None:
                    tap = tap * (row_m[di] * col_m[dj])
                elif row_m[di] is not None:
                    tap = tap * row_m[di]
                elif col_m[dj] is not None:
                    tap = tap * col_m[dj]
                p_ref[t * c_in:(t + 1) * c_in, :] = tap

        pv = p_ref[...]
        p2_ref[...] = pv * pv
        mu = jnp.dot(wm_ref[...], p_ref[...],
                     preferred_element_type=jnp.float32)
        var = jnp.dot(wv_ref[...], p2_ref[...],
                      preferred_element_type=jnp.float32)

        y = mu + jnp.sqrt(_VAR_EPS + var) * noise_ref[0]
        yb = y.astype(jnp.bfloat16)
        y_scr[i] = yb
        y32 = yb.astype(jnp.float32)
        s = jnp.sum(y32, axis=1, keepdims=True)
        sq = jnp.sum(y32 * y32, axis=1, keepdims=True)

        @pl.when(i == 0)
        def _():
            sum_scr[...] = s
            sq_scr[...] = sq

        @pl.when(i > 0)
        def _():
            sum_scr[...] += s
            sq_scr[...] += sq

    @pl.when(i == n_img)
    def _finalize_stats():
        m = n_img * hw
        mean = sum_scr[...] * (1.0 / m)
        var = sq_scr[...] * (1.0 / m) - mean * mean
        inv = g_ref[...] * jax.lax.rsqrt(var + _BN_EPS)
        scale_scr[...] = inv
        shift_scr[...] = b_ref[...] - mean * inv

    @pl.when(i >= n_img)
    def _apply_phase():
        im = i - n_img
        o_ref[0] = (y_scr[im].astype(jnp.float32) * scale_scr[...]
                    + shift_scr[...])


def kernel(x, weight, log_sigma, gamma, beta, noise):
    n, c_in, h, w = x.shape
    c_out, _, kh, kw = weight.shape
    ho, wo = h, w
    hw = ho * wo
    k = c_in * kh * kw

    x3 = x.reshape(n, c_in, hw)
    noise3 = noise.reshape(n, c_out, hw)
    wm = weight.transpose(0, 2, 3, 1).reshape(c_out, k).astype(jnp.bfloat16)
    wv = (0.01 + jnp.exp(2.0 * log_sigma.astype(jnp.float32)))
    wv = wv.transpose(0, 2, 3, 1).reshape(c_out, k).astype(jnp.bfloat16)
    g2 = gamma.reshape(c_out, 1).astype(jnp.float32)
    b2 = beta.reshape(c_out, 1).astype(jnp.float32)

    last = n - 1
    out3 = pl.pallas_call(
        functools.partial(_fused_kernel, n_img=n, kh=kh, kw=kw, c_in=c_in,
                          ho_dim=ho, wo_dim=wo),
        grid=(2 * n,),
        in_specs=[
            pl.BlockSpec((1, c_in, hw),
                         lambda i: (jnp.minimum(i, last), 0, 0)),
            pl.BlockSpec((c_out, k), lambda i: (0, 0)),
            pl.BlockSpec((c_out, k), lambda i: (0, 0)),
            pl.BlockSpec((1, c_out, hw),
                         lambda i: (jnp.minimum(i, last), 0, 0)),
            pl.BlockSpec((c_out, 1), lambda i: (0, 0)),
            pl.BlockSpec((c_out, 1), lambda i: (0, 0)),
        ],
        out_specs=pl.BlockSpec((1, c_out, hw),
                               lambda i: (jnp.maximum(i - (last + 1), 0),
                                          0, 0)),
        out_shape=jax.ShapeDtypeStruct((n, c_out, hw), jnp.float32),
        scratch_shapes=[
            pltpu.VMEM((n, c_out, hw), jnp.bfloat16),
            pltpu.VMEM((k, hw), jnp.bfloat16),
            pltpu.VMEM((k, hw), jnp.bfloat16),
            pltpu.VMEM((c_out, 1), jnp.float32),
            pltpu.VMEM((c_out, 1), jnp.float32),
            pltpu.VMEM((c_out, 1), jnp.float32),
            pltpu.VMEM((c_out, 1), jnp.float32),
        ],
        compiler_params=pltpu.CompilerParams(
            dimension_semantics=("arbitrary",),
            vmem_limit_bytes=100 * 1024 * 1024,
        ),
    )(x3, wm, wv, noise3, g2, b2)

    return out3.reshape(n, c_out, ho, wo)
